# baseline (speedup 1.0000x reference)
import jax
import jax.numpy as jnp
from jax.experimental import pallas as pl
from jax.experimental.pallas import tpu as pltpu

N = 4096
D = 128
H = 32
Z = 16
CPAD = 128

BM = 512
BK = 2048
NI = N // BM
NK = N // BK

P0 = NI
P1 = P0 + NI * NK
P2 = P1 + NI * NK
P3 = P2 + NI * NK
P4 = P3 + NI * NK
P5 = P4 + NI * NK


def _phase_ik(s, lo):
    t = s - lo
    return t // NK, t % NK


def _split_dot(r, y):
    rb = r.astype(jnp.bfloat16)
    hi = y.astype(jnp.bfloat16)
    lo = (y - hi.astype(jnp.float32)).astype(jnp.bfloat16)
    return (jnp.dot(rb, hi, preferred_element_type=jnp.float32) +
            jnp.dot(rb, lo, preferred_element_type=jnp.float32))


def _mega_kernel(adj_ref, f_ref, wb_ref, w0_ref, wm_ref, b0_ref, w1_ref,
                 b1_ref, ncp_ref, logits_ref,
                 p1_s, x3_s, p2_s, mean_s, y4_s, dvec_s, rdiag_s,
                 acc32, acc16, acc128, dacc, rd_s, mmax_s):
    s = pl.program_id(0)

    @pl.when(s < P0)
    def _():
        f = f_ref[...]
        p1_s[pl.ds(s * BM, BM), :] = jnp.dot(
            f, wb_ref[...], preferred_element_type=jnp.float32)
        x3_s[pl.ds(s * BM, BM), :] = jnp.dot(
            f, w0_ref[...], preferred_element_type=jnp.float32)

    @pl.when((s >= P0) & (s < P1))
    def _():
        i, k = _phase_ik(s, P0)

        @pl.when(k == 0)
        def _():
            acc32[...] = jnp.zeros_like(acc32)

        acc32[...] += jnp.dot(adj_ref[...], p1_s[pl.ds(k * BK, BK), :],
                              preferred_element_type=jnp.float32)

        @pl.when(k == NK - 1)
        def _():
            p2_s[pl.ds(i * BM, BM), :] = jnp.dot(
                acc32[...], wm_ref[...], preferred_element_type=jnp.float32)

    @pl.when((s >= P1) & (s < P2))
    def _():
        i, k = _phase_ik(s, P1)

        @pl.when(k == 0)
        def _():
            acc16[...] = jnp.zeros_like(acc16)

        acc16[...] += jnp.dot(adj_ref[...], p2_s[pl.ds(k * BK, BK), :],
                              preferred_element_type=jnp.float32)

        @pl.when(k == NK - 1)
        def _():
            m = jnp.maximum(acc16[...], 0.0)
            mean_s[pl.ds(i * BM, BM), :] = m
            blk = jnp.max(jnp.sum(m * m, axis=1))

            @pl.when(i == 0)
            def _():
                mmax_s[0, 0] = blk

            mmax_s[0, 0] = jnp.maximum(mmax_s[0, 0], blk)

    @pl.when((s >= P2) & (s < P3))
    def _():
        i, j = _phase_ik(s, P2)

        @pl.when(j == 0)
        def _():
            dacc[...] = jnp.zeros_like(dacc)

        mi = mean_s[pl.ds(i * BM, BM), :]
        mj = mean_s[pl.ds(j * BK, BK), :]
        g = jax.lax.dot_general(mi, mj, (((1,), (1,)), ((), ())),
                                preferred_element_type=jnp.float32)
        logits_ref[...] = g
        r = jnp.round(g * (1.0 / mmax_s[0, 0]))
        dacc[...] += jnp.dot(r, jnp.ones((BK, 1), jnp.float32),
                             preferred_element_type=jnp.float32)

        @pl.when(j == (i * BM) // BK)
        def _():
            rows = jax.lax.broadcasted_iota(jnp.int32, (BM, BK), 0) + i * BM
            cols = jax.lax.broadcasted_iota(jnp.int32, (BM, BK), 1) + j * BK
            rd_s[...] = jnp.dot(jnp.where(rows == cols, r, 0.0),
                                jnp.ones((BK, 1), jnp.float32),
                                preferred_element_type=jnp.float32)

        @pl.when(j == NK - 1)
        def _():
            rd = rd_s[...]
            rdiag_s[pl.ds(i * BM, BM), :] = rd
            dvec_s[pl.ds(i * BM, BM), :] = jax.lax.pow(
                dacc[...] + (1.0 - rd), -0.5)

    @pl.when((s >= P3) & (s < P4))
    def _():
        i, k = _phase_ik(s, P3)

        @pl.when(k == 0)
        def _():
            acc32[...] = jnp.zeros_like(acc32)

        mi = mean_s[pl.ds(i * BM, BM), :]
        mk = mean_s[pl.ds(k * BK, BK), :]
        g = jax.lax.dot_general(mi, mk, (((1,), (1,)), ((), ())),
                                preferred_element_type=jnp.float32)
        r = jnp.round(g * (1.0 / mmax_s[0, 0]))
        y3 = dvec_s[pl.ds(k * BK, BK), :] * x3_s[pl.ds(k * BK, BK), :]
        acc32[...] += _split_dot(r, y3)

        @pl.when(k == NK - 1)
        def _():
            di = dvec_s[pl.ds(i * BM, BM), :]
            fix = (1.0 - rdiag_s[pl.ds(i * BM, BM), :])
            acc = acc32[...] + fix * (di * x3_s[pl.ds(i * BM, BM), :])
            h = jnp.maximum(di * acc + b0_ref[...], 0.0)
            y4_s[pl.ds(i * BM, BM), :] = di * jnp.dot(
                h, w1_ref[...], preferred_element_type=jnp.float32)

    @pl.when(s >= P4)
    def _():
        i, k = _phase_ik(s, P4)

        @pl.when(k == 0)
        def _():
            acc128[...] = jnp.zeros_like(acc128)

        mi = mean_s[pl.ds(i * BM, BM), :]
        mk = mean_s[pl.ds(k * BK, BK), :]
        g = jax.lax.dot_general(mi, mk, (((1,), (1,)), ((), ())),
                                preferred_element_type=jnp.float32)
        r = jnp.round(g * (1.0 / mmax_s[0, 0]))
        acc128[...] += _split_dot(r, y4_s[pl.ds(k * BK, BK), :])

        @pl.when(k == NK - 1)
        def _():
            di = dvec_s[pl.ds(i * BM, BM), :]
            fix = (1.0 - rdiag_s[pl.ds(i * BM, BM), :])
            acc = acc128[...] + fix * y4_s[pl.ds(i * BM, BM), :]
            ncp_ref[...] = di * acc + b1_ref[...]


def _clampi(x, hi):
    return jnp.minimum(x, hi)


def _adj_idx(s):
    in1 = (s >= P0) & (s < P1)
    in2 = (s >= P1) & (s < P2)
    t = jnp.where(in1, s - P0, jnp.where(in2, s - P1, (NI * NK) - 1))
    return t // NK, t % NK


def _feat_idx(s):
    return _clampi(s, NI - 1), 0


def _logits_idx(s):
    t = jnp.clip(s - P2, 0, NI * NK - 1)
    return t // NK, t % NK


def _ncp_idx(s):
    t = jnp.clip(s - P4, 0, NI * NK - 1)
    return t // NK, 0


@jax.jit
def kernel(adj, adj_orig, features, W_base, W_mean, W_nc0, b_nc0, W_nc1,
                b_nc1):
    del adj_orig
    f32 = jnp.float32
    b0 = b_nc0.reshape(1, H)
    w1p = jnp.zeros((H, CPAD), f32).at[:, :b_nc1.shape[0]].set(W_nc1)
    b1p = jnp.zeros((1, CPAD), f32).at[0, :b_nc1.shape[0]].set(b_nc1)
    const = lambda a, b: (lambda s: (a, b))

    ncp, adj_logits = pl.pallas_call(
        _mega_kernel,
        grid=(P5,),
        in_specs=[
            pl.BlockSpec((BM, BK), _adj_idx),
            pl.BlockSpec((BM, D), _feat_idx),
            pl.BlockSpec((D, H), const(0, 0)),
            pl.BlockSpec((D, H), const(0, 0)),
            pl.BlockSpec((H, Z), const(0, 0)),
            pl.BlockSpec((1, H), const(0, 0)),
            pl.BlockSpec((H, CPAD), const(0, 0)),
            pl.BlockSpec((1, CPAD), const(0, 0)),
        ],
        out_specs=[
            pl.BlockSpec((BM, CPAD), _ncp_idx),
            pl.BlockSpec((BM, BK), _logits_idx),
        ],
        out_shape=[
            jax.ShapeDtypeStruct((N, CPAD), f32),
            jax.ShapeDtypeStruct((N, N), f32),
        ],
        scratch_shapes=[
            pltpu.VMEM((N, H), f32),
            pltpu.VMEM((N, H), f32),
            pltpu.VMEM((N, Z), f32),
            pltpu.VMEM((N, Z), f32),
            pltpu.VMEM((N, CPAD), f32),
            pltpu.VMEM((N, 1), f32),
            pltpu.VMEM((N, 1), f32),
            pltpu.VMEM((BM, H), f32),
            pltpu.VMEM((BM, Z), f32),
            pltpu.VMEM((BM, CPAD), f32),
            pltpu.VMEM((BM, 1), f32),
            pltpu.VMEM((BM, 1), f32),
            pltpu.SMEM((1, 1), f32),
        ],
    )(adj, features, W_base, W_nc0, W_mean, b0, w1p, b1p)

    return (ncp[:, :b_nc1.shape[0]], adj_logits)


# speedup vs baseline: 1.2023x; 1.2023x over previous
import jax
import jax.numpy as jnp
from jax.experimental import pallas as pl
from jax.experimental.pallas import tpu as pltpu

N = 4096
D = 128
H = 32
Z = 16
CPAD = 128

BM = 512
BK = 2048
NI = N // BM
NK = N // BK

P0 = NI
P1 = P0 + NI
P2 = P1 + NI
P3 = P2 + NI * NK
P4 = P3 + NI * NK
P5 = P4 + NI * NK


def _phase_ik(s, lo):
    t = s - lo
    return t // NK, t % NK


def _split_dot(r, y):
    rb = r.astype(jnp.bfloat16)
    hi = y.astype(jnp.bfloat16)
    lo = (y - hi.astype(jnp.float32)).astype(jnp.bfloat16)
    return (jnp.dot(rb, hi, preferred_element_type=jnp.float32) +
            jnp.dot(rb, lo, preferred_element_type=jnp.float32))


def _mega_kernel(adj_ref, f_ref, wb_ref, w0_ref, wm_ref, b0_ref, w1_ref,
                 b1_ref, ncp_ref, logits_ref,
                 p1_s, x3_s, p2_s, mean_s, y4_s, dvec_s, rdiag_s,
                 acc32, acc16, acc128, dacc, rd_s, mmax_s):
    s = pl.program_id(0)

    @pl.when(s < P0)
    def _():
        f = f_ref[...]
        p1_s[pl.ds(s * BM, BM), :] = jnp.dot(
            f, wb_ref[...], preferred_element_type=jnp.float32)
        x3_s[pl.ds(s * BM, BM), :] = jnp.dot(
            f, w0_ref[...], preferred_element_type=jnp.float32)

    @pl.when((s >= P0) & (s < P1))
    def _():
        i = s - P0
        q = jnp.dot(adj_ref[...], p1_s[...],
                    preferred_element_type=jnp.float32)
        p2_s[pl.ds(i * BM, BM), :] = jnp.dot(
            q, wm_ref[...], preferred_element_type=jnp.float32)

    @pl.when((s >= P1) & (s < P2))
    def _():
        i = s - P1
        m = jnp.maximum(jnp.dot(adj_ref[...], p2_s[...],
                                preferred_element_type=jnp.float32), 0.0)
        mean_s[pl.ds(i * BM, BM), :] = m
        blk = jnp.max(jnp.sum(m * m, axis=1))

        @pl.when(i == 0)
        def _():
            mmax_s[0, 0] = blk

        mmax_s[0, 0] = jnp.maximum(mmax_s[0, 0], blk)

    @pl.when((s >= P2) & (s < P3))
    def _():
        i, j = _phase_ik(s, P2)

        @pl.when(j == 0)
        def _():
            dacc[...] = jnp.zeros_like(dacc)

        mi = mean_s[pl.ds(i * BM, BM), :]
        mj = mean_s[pl.ds(j * BK, BK), :]
        g = jax.lax.dot_general(mi, mj, (((1,), (1,)), ((), ())),
                                preferred_element_type=jnp.float32)
        logits_ref[...] = g
        r = jnp.round(g * (1.0 / mmax_s[0, 0]))
        dacc[...] += jnp.dot(r, jnp.ones((BK, 1), jnp.float32),
                             preferred_element_type=jnp.float32)

        @pl.when(j == (i * BM) // BK)
        def _():
            rows = jax.lax.broadcasted_iota(jnp.int32, (BM, BK), 0) + i * BM
            cols = jax.lax.broadcasted_iota(jnp.int32, (BM, BK), 1) + j * BK
            rd_s[...] = jnp.dot(jnp.where(rows == cols, r, 0.0),
                                jnp.ones((BK, 1), jnp.float32),
                                preferred_element_type=jnp.float32)

        @pl.when(j == NK - 1)
        def _():
            rd = rd_s[...]
            rdiag_s[pl.ds(i * BM, BM), :] = rd
            dvec_s[pl.ds(i * BM, BM), :] = jax.lax.pow(
                dacc[...] + (1.0 - rd), -0.5)

    @pl.when((s >= P3) & (s < P4))
    def _():
        i, k = _phase_ik(s, P3)

        @pl.when(k == 0)
        def _():
            acc32[...] = jnp.zeros_like(acc32)

        mi = mean_s[pl.ds(i * BM, BM), :]
        mk = mean_s[pl.ds(k * BK, BK), :]
        g = jax.lax.dot_general(mi, mk, (((1,), (1,)), ((), ())),
                                preferred_element_type=jnp.float32)
        r = jnp.round(g * (1.0 / mmax_s[0, 0]))
        y3 = dvec_s[pl.ds(k * BK, BK), :] * x3_s[pl.ds(k * BK, BK), :]
        acc32[...] += jnp.dot(r, y3, preferred_element_type=jnp.float32)

        @pl.when(k == NK - 1)
        def _():
            di = dvec_s[pl.ds(i * BM, BM), :]
            fix = (1.0 - rdiag_s[pl.ds(i * BM, BM), :])
            acc = acc32[...] + fix * (di * x3_s[pl.ds(i * BM, BM), :])
            h = jnp.maximum(di * acc + b0_ref[...], 0.0)
            y4_s[pl.ds(i * BM, BM), :] = di * jnp.dot(
                h, w1_ref[...], preferred_element_type=jnp.float32)

    @pl.when(s >= P4)
    def _():
        i, k = _phase_ik(s, P4)

        @pl.when(k == 0)
        def _():
            acc128[...] = jnp.zeros_like(acc128)

        mi = mean_s[pl.ds(i * BM, BM), :]
        mk = mean_s[pl.ds(k * BK, BK), :]
        g = jax.lax.dot_general(mi, mk, (((1,), (1,)), ((), ())),
                                preferred_element_type=jnp.float32)
        r = jnp.round(g * (1.0 / mmax_s[0, 0]))
        acc128[...] += jnp.dot(r, y4_s[pl.ds(k * BK, BK), :],
                               preferred_element_type=jnp.float32)

        @pl.when(k == NK - 1)
        def _():
            di = dvec_s[pl.ds(i * BM, BM), :]
            fix = (1.0 - rdiag_s[pl.ds(i * BM, BM), :])
            acc = acc128[...] + fix * y4_s[pl.ds(i * BM, BM), :]
            ncp_ref[...] = di * acc + b1_ref[...]


def _clampi(x, hi):
    return jnp.minimum(x, hi)


def _adj_idx(s):
    in1 = (s >= P0) & (s < P1)
    in2 = (s >= P1) & (s < P2)
    t = jnp.where(in1, s - P0, jnp.where(in2, s - P1, NI - 1))
    return t, 0


def _feat_idx(s):
    return _clampi(s, NI - 1), 0


def _logits_idx(s):
    t = jnp.clip(s - P2, 0, NI * NK - 1)
    return t // NK, t % NK


def _ncp_idx(s):
    t = jnp.clip(s - P4, 0, NI * NK - 1)
    return t // NK, 0


@jax.jit
def kernel(adj, adj_orig, features, W_base, W_mean, W_nc0, b_nc0, W_nc1,
                b_nc1):
    del adj_orig
    f32 = jnp.float32
    b0 = b_nc0.reshape(1, H)
    w1p = jnp.zeros((H, CPAD), f32).at[:, :b_nc1.shape[0]].set(W_nc1)
    b1p = jnp.zeros((1, CPAD), f32).at[0, :b_nc1.shape[0]].set(b_nc1)
    const = lambda a, b: (lambda s: (a, b))

    ncp, adj_logits = pl.pallas_call(
        _mega_kernel,
        grid=(P5,),
        in_specs=[
            pl.BlockSpec((BM, N), _adj_idx),
            pl.BlockSpec((BM, D), _feat_idx),
            pl.BlockSpec((D, H), const(0, 0)),
            pl.BlockSpec((D, H), const(0, 0)),
            pl.BlockSpec((H, Z), const(0, 0)),
            pl.BlockSpec((1, H), const(0, 0)),
            pl.BlockSpec((H, CPAD), const(0, 0)),
            pl.BlockSpec((1, CPAD), const(0, 0)),
        ],
        out_specs=[
            pl.BlockSpec((BM, CPAD), _ncp_idx),
            pl.BlockSpec((BM, BK), _logits_idx),
        ],
        out_shape=[
            jax.ShapeDtypeStruct((N, CPAD), f32),
            jax.ShapeDtypeStruct((N, N), f32),
        ],
        scratch_shapes=[
            pltpu.VMEM((N, H), f32),
            pltpu.VMEM((N, H), f32),
            pltpu.VMEM((N, Z), f32),
            pltpu.VMEM((N, Z), f32),
            pltpu.VMEM((N, CPAD), f32),
            pltpu.VMEM((N, 1), f32),
            pltpu.VMEM((N, 1), f32),
            pltpu.VMEM((BM, H), f32),
            pltpu.VMEM((BM, Z), f32),
            pltpu.VMEM((BM, CPAD), f32),
            pltpu.VMEM((BM, 1), f32),
            pltpu.VMEM((BM, 1), f32),
            pltpu.SMEM((1, 1), f32),
        ],
    )(adj, features, W_base, W_nc0, W_mean, b0, w1p, b1p)

    return (ncp[:, :b_nc1.shape[0]], adj_logits)


# speedup vs baseline: 1.3012x; 1.0822x over previous
import jax
import jax.numpy as jnp
from jax.experimental import pallas as pl
from jax.experimental.pallas import tpu as pltpu

N = 4096
D = 128
H = 32
Z = 16
CPAD = 128

BM = 512
BKG = 1024
BK = 2048
NI = N // BM
NKG = N // BKG
NK = N // BK

P0 = NI
P1 = P0 + NI
P2 = P1 + NI
P3 = P2 + NI * NKG
P4 = P3 + NI * NK
P5 = P4 + NI * NK


def _mega_kernel(adj_ref, f_ref, wb_ref, w0_ref, wm_ref, b0_ref, w1_ref,
                 b1_ref, ncp_ref, logits_ref,
                 p1_s, x3_s, p2_s, mean_s, y4_s, dvec_s, rdiag_s,
                 acc32, acc128, dacc, rd_s, mmax_s, r8_s):
    s = pl.program_id(0)

    @pl.when(s < P0)
    def _():
        f = f_ref[...]
        p1_s[pl.ds(s * BM, BM), :] = jnp.dot(
            f, wb_ref[...], preferred_element_type=jnp.float32)
        x3_s[pl.ds(s * BM, BM), :] = jnp.dot(
            f, w0_ref[...], preferred_element_type=jnp.float32)

    @pl.when((s >= P0) & (s < P1))
    def _():
        i = s - P0
        q = jnp.dot(adj_ref[...], p1_s[...],
                    preferred_element_type=jnp.float32)
        p2_s[pl.ds(i * BM, BM), :] = jnp.dot(
            q, wm_ref[...], preferred_element_type=jnp.float32)

    @pl.when((s >= P1) & (s < P2))
    def _():
        i = s - P1
        m = jnp.maximum(jnp.dot(adj_ref[...], p2_s[...],
                                preferred_element_type=jnp.float32), 0.0)
        mean_s[pl.ds(i * BM, BM), :] = m
        blk = jnp.max(jnp.sum(m * m, axis=1))

        @pl.when(i == 0)
        def _():
            mmax_s[0, 0] = blk

        mmax_s[0, 0] = jnp.maximum(mmax_s[0, 0], blk)

    @pl.when((s >= P2) & (s < P3))
    def _():
        t = s - P2
        i = t // NKG
        j = t % NKG

        @pl.when(j == 0)
        def _():
            dacc[...] = jnp.zeros_like(dacc)

        mi = mean_s[pl.ds(i * BM, BM), :]
        mj = mean_s[pl.ds(j * BKG, BKG), :]
        g = jax.lax.dot_general(mi, mj, (((1,), (1,)), ((), ())),
                                preferred_element_type=jnp.float32)
        logits_ref[...] = g
        r = jnp.round(g * (1.0 / mmax_s[0, 0]))
        r8_s[pl.ds(i * BM, BM), pl.ds(j * BKG, BKG)] = r.astype(jnp.int8)
        dacc[...] += jnp.dot(r, jnp.ones((BKG, 1), jnp.float32),
                             preferred_element_type=jnp.float32)

        @pl.when(j == (i * BM) // BKG)
        def _():
            rows = jax.lax.broadcasted_iota(jnp.int32, (BM, BKG), 0) + i * BM
            cols = jax.lax.broadcasted_iota(jnp.int32, (BM, BKG), 1) + j * BKG
            rd_s[...] = jnp.dot(jnp.where(rows == cols, r, 0.0),
                                jnp.ones((BKG, 1), jnp.float32),
                                preferred_element_type=jnp.float32)

        @pl.when(j == NKG - 1)
        def _():
            rd = rd_s[...]
            rdiag_s[pl.ds(i * BM, BM), :] = rd
            dvec_s[pl.ds(i * BM, BM), :] = jax.lax.pow(
                dacc[...] + (1.0 - rd), -0.5)

    @pl.when((s >= P3) & (s < P4))
    def _():
        t = s - P3
        i = t // NK
        k = t % NK

        @pl.when(k == 0)
        def _():
            acc32[...] = jnp.zeros_like(acc32)

        r = r8_s[pl.ds(i * BM, BM), pl.ds(k * BK, BK)].astype(jnp.float32)
        y3 = dvec_s[pl.ds(k * BK, BK), :] * x3_s[pl.ds(k * BK, BK), :]
        acc32[...] += jnp.dot(r, y3, preferred_element_type=jnp.float32)

        @pl.when(k == NK - 1)
        def _():
            di = dvec_s[pl.ds(i * BM, BM), :]
            fix = (1.0 - rdiag_s[pl.ds(i * BM, BM), :])
            acc = acc32[...] + fix * (di * x3_s[pl.ds(i * BM, BM), :])
            h = jnp.maximum(di * acc + b0_ref[...], 0.0)
            y4_s[pl.ds(i * BM, BM), :] = di * jnp.dot(
                h, w1_ref[...], preferred_element_type=jnp.float32)

    @pl.when(s >= P4)
    def _():
        t = s - P4
        i = t // NK
        k = t % NK

        @pl.when(k == 0)
        def _():
            acc128[...] = jnp.zeros_like(acc128)

        r = r8_s[pl.ds(i * BM, BM), pl.ds(k * BK, BK)].astype(jnp.float32)
        acc128[...] += jnp.dot(r, y4_s[pl.ds(k * BK, BK), :],
                               preferred_element_type=jnp.float32)

        @pl.when(k == NK - 1)
        def _():
            di = dvec_s[pl.ds(i * BM, BM), :]
            fix = (1.0 - rdiag_s[pl.ds(i * BM, BM), :])
            acc = acc128[...] + fix * y4_s[pl.ds(i * BM, BM), :]
            ncp_ref[...] = di * acc + b1_ref[...]


def _adj_idx(s):
    in1 = (s >= P0) & (s < P1)
    in2 = (s >= P1) & (s < P2)
    t = jnp.where(in1, s - P0, jnp.where(in2, s - P1, NI - 1))
    return t, 0


def _feat_idx(s):
    return jnp.minimum(s, NI - 1), 0


def _logits_idx(s):
    t = jnp.clip(s - P2, 0, NI * NKG - 1)
    return t // NKG, t % NKG


def _ncp_idx(s):
    t = jnp.clip(s - P4, 0, NI * NK - 1)
    return t // NK, 0


@jax.jit
def kernel(adj, adj_orig, features, W_base, W_mean, W_nc0, b_nc0, W_nc1,
                b_nc1):
    del adj_orig
    f32 = jnp.float32
    b0 = b_nc0.reshape(1, H)
    w1p = jnp.zeros((H, CPAD), f32).at[:, :b_nc1.shape[0]].set(W_nc1)
    b1p = jnp.zeros((1, CPAD), f32).at[0, :b_nc1.shape[0]].set(b_nc1)
    const = lambda a, b: (lambda s: (a, b))

    ncp, adj_logits = pl.pallas_call(
        _mega_kernel,
        grid=(P5,),
        in_specs=[
            pl.BlockSpec((BM, N), _adj_idx),
            pl.BlockSpec((BM, D), _feat_idx),
            pl.BlockSpec((D, H), const(0, 0)),
            pl.BlockSpec((D, H), const(0, 0)),
            pl.BlockSpec((H, Z), const(0, 0)),
            pl.BlockSpec((1, H), const(0, 0)),
            pl.BlockSpec((H, CPAD), const(0, 0)),
            pl.BlockSpec((1, CPAD), const(0, 0)),
        ],
        out_specs=[
            pl.BlockSpec((BM, CPAD), _ncp_idx),
            pl.BlockSpec((BM, BKG), _logits_idx),
        ],
        out_shape=[
            jax.ShapeDtypeStruct((N, CPAD), f32),
            jax.ShapeDtypeStruct((N, N), f32),
        ],
        scratch_shapes=[
            pltpu.VMEM((N, H), f32),
            pltpu.VMEM((N, H), f32),
            pltpu.VMEM((N, Z), f32),
            pltpu.VMEM((N, Z), f32),
            pltpu.VMEM((N, CPAD), f32),
            pltpu.VMEM((N, 1), f32),
            pltpu.VMEM((N, 1), f32),
            pltpu.VMEM((BM, H), f32),
            pltpu.VMEM((BM, CPAD), f32),
            pltpu.VMEM((BM, 1), f32),
            pltpu.VMEM((BM, 1), f32),
            pltpu.SMEM((1, 1), f32),
            pltpu.VMEM((N, N), jnp.int8),
        ],
    )(adj, features, W_base, W_nc0, W_mean, b0, w1p, b1p)

    return (ncp[:, :b_nc1.shape[0]], adj_logits)
